# baseline (speedup 1.0000x reference)
import functools

import jax
import jax.numpy as jnp
from jax import lax
from jax.experimental import pallas as pl
from jax.experimental.pallas import tpu as pltpu
from jax.experimental.pallas import tpu_sc as plsc

N = 10000
E = 160000
G = 300
H2 = 100
R = 8
NBASES = 30
NT = R + 1

L = 16
NS = 16
NC = 2
DQ = 112
CH = 128
NCHG = E // CH
TPC1 = -(-NCHG // NS)
NP1 = 40
NW2 = NC * NS
NP2 = 20
DB = 80
NDC = N // DB
DPT = -(-NDC // NS)
CNT = R * N
CZB = 1000
BN = 1000


def _wcomp_body(comp_ref, basis_ref, out_ref):
    out_ref[...] = jnp.dot(comp_ref[...], basis_ref[...],
                           preferred_element_type=jnp.float32)


def _wp_body(w_ref, root_ref, b_ref, pj_ref, out_ref, bp_ref):
    t = pl.program_id(0) % NT

    @pl.when(t < R)
    def _():
        out_ref[0] = jnp.dot(w_ref[0], pj_ref[0],
                             preferred_element_type=jnp.float32)

    @pl.when(t == R)
    def _():
        out_ref[0] = jnp.dot(root_ref[...], pj_ref[0],
                             preferred_element_type=jnp.float32)

    bp_ref[0] = jnp.dot(b_ref[...], pj_ref[0],
                        preferred_element_type=jnp.float32)


def _htab_body(x_ref, w_ref, out_ref):
    out_ref[0] = jnp.dot(x_ref[...], w_ref[0],
                         preferred_element_type=jnp.float32)


def _final_body(a0_ref, a1_ref, o1_ref, b_ref, out_ref):
    acc = a0_ref[0] + a1_ref[0] + o1_ref[0]
    out_ref[...] = acc[:, :H2] + b_ref[0]


_MESH = plsc.VectorSubcoreMesh(core_axis_name="c", subcore_axis_name="s",
                               num_cores=NC, num_subcores=NS)
_SC_PARAMS = pltpu.CompilerParams(use_tc_tiling_on_sc=False)


def _conv1_sc(src_hbm, dst_hbm, typ_hbm, h_hbm, bp_hbm, z2d_hbm, z1d_hbm,
              out_hbm,
              srcv0, srcv1, dstv0, dstv1, typv0, typv1, idxv0, idxv1,
              keyv0, keyv1, cntv0, cntv1, onesv, rows0, rows1, z1, bpv,
              agg_sh, cnt_sh, sem_ld0, sem_ld1, sem_cg0, sem_cg1,
              sem_g0, sem_g1):
    c = lax.axis_index("c")
    s = lax.axis_index("s")
    coff = c * (NT * N)

    bufs = (
        (srcv0, dstv0, typv0, idxv0, keyv0, cntv0, rows0,
         sem_ld0, sem_cg0, sem_g0),
        (srcv1, dstv1, typv1, idxv1, keyv1, cntv1, rows1,
         sem_ld1, sem_cg1, sem_g1),
    )

    pltpu.sync_copy(z2d_hbm, rows0.at[pl.ds(0, DB), :])
    pltpu.sync_copy(z1d_hbm, z1)
    for b in range(DPT):
        g = b * NS + s

        @pl.when(g < NDC)
        def _():
            st = pl.multiple_of(g * DB, 8)
            pltpu.sync_copy(rows0.at[pl.ds(0, DB), :],
                            agg_sh.at[pl.ds(st, DB), :])
    for b in range(5):
        st = pl.multiple_of(s * (5 * CZB) + b * CZB, 8)
        pltpu.sync_copy(z1, cnt_sh.at[pl.ds(st, CZB)])

    def _fill_ones(i, carry):
        onesv[pl.ds(i * L, L)] = jnp.full((L,), 1.0, jnp.float32)
        return carry
    lax.fori_loop(0, CH // L, _fill_ones, 0)
    plsc.subcore_barrier()

    kbufs = ((dstv0, typv0, keyv0, sem_ld0),
             (dstv1, typv1, keyv1, sem_ld1))

    def _cstage_a(k, b):
        g = k * NS + s

        @pl.when(g < NCHG)
        def _():
            dstv, typv, keyv, s_ld = kbufs[b]
            off = g * CH
            pltpu.async_copy(dst_hbm.at[pl.ds(off, CH)], dstv, s_ld)
            pltpu.async_copy(typ_hbm.at[pl.ds(off, CH)], typv, s_ld)

    def _cstage_p(k, b):
        g = k * NS + s

        @pl.when(g < NCHG)
        def _():
            dstv, typv, keyv, s_ld = kbufs[b]
            off = g * CH
            pltpu.make_async_copy(dst_hbm.at[pl.ds(off, CH)], dstv,
                                  s_ld).wait()
            pltpu.make_async_copy(typ_hbm.at[pl.ds(off, CH)], typv,
                                  s_ld).wait()

            def _keys(i, carry2):
                sl = pl.ds(i * L, L)
                keyv[sl] = typv[sl] * N + dstv[sl]
                return carry2
            lax.fori_loop(0, CH // L, _keys, 0)
            pltpu.sync_copy(onesv, cnt_sh.at[keyv], add=True)

    _cstage_a(0, 0)
    _cstage_a(1, 1)

    def _cpipe(k2, carry):
        base = 2 * k2
        _cstage_p(base, 0)
        _cstage_a(base + 2, 0)
        _cstage_p(base + 1, 1)
        _cstage_a(base + 3, 1)
        return carry
    lax.fori_loop(0, NP1, _cpipe, 0)
    plsc.subcore_barrier()

    def _stage_a(k, b):
        g = k * NS + s

        @pl.when(g < NCHG)
        def _():
            srcv, dstv, typv, idxv, keyv, cntv, rows, s_ld, s_cg, s_g = \
                bufs[b]
            off = g * CH
            pltpu.async_copy(src_hbm.at[pl.ds(off, CH)], srcv, s_ld)
            pltpu.async_copy(dst_hbm.at[pl.ds(off, CH)], dstv, s_ld)
            pltpu.async_copy(typ_hbm.at[pl.ds(off, CH)], typv, s_ld)

    def _stage_g(k, b):
        g = k * NS + s

        @pl.when(g < NCHG)
        def _():
            srcv, dstv, typv, idxv, keyv, cntv, rows, s_ld, s_cg, s_g = \
                bufs[b]
            off = g * CH
            pltpu.make_async_copy(src_hbm.at[pl.ds(off, CH)], srcv,
                                  s_ld).wait()
            pltpu.make_async_copy(dst_hbm.at[pl.ds(off, CH)], dstv,
                                  s_ld).wait()
            pltpu.make_async_copy(typ_hbm.at[pl.ds(off, CH)], typv,
                                  s_ld).wait()

            def _keys(i, carry2):
                sl = pl.ds(i * L, L)
                t = typv[sl]
                keyv[sl] = t * N + dstv[sl]
                idxv[sl] = coff + t * N + srcv[sl]
                return carry2
            lax.fori_loop(0, CH // L, _keys, 0)
            pltpu.async_copy(cnt_sh.at[keyv], cntv, s_cg)
            pltpu.async_copy(h_hbm.at[idxv], rows, s_g)

    def _stage_p(k, b):
        g = k * NS + s

        @pl.when(g < NCHG)
        def _():
            srcv, dstv, typv, idxv, keyv, cntv, rows, s_ld, s_cg, s_g = \
                bufs[b]
            pltpu.make_async_copy(cnt_sh.at[keyv], cntv, s_cg).wait()
            pltpu.make_async_copy(h_hbm.at[idxv], rows, s_g).wait()

            def _mul(i, carry2):
                cnt16 = cntv[pl.ds(i * L, L)]
                sc = 1.0 / jnp.maximum(cnt16, 1.0)
                for j2 in range(L):
                    s16 = jnp.take_along_axis(
                        sc, jnp.full((L,), j2, jnp.int32), axis=0)
                    row = i * L + j2
                    for v in range(DQ // L):
                        sl = pl.ds(v * L, L)
                        rows[row, sl] = rows[row, sl] * s16
                return carry2
            lax.fori_loop(0, CH // L, _mul, 0)
            pltpu.sync_copy(rows, agg_sh.at[dstv], add=True)

    _stage_a(0, 0)
    _stage_a(1, 1)
    _stage_g(0, 0)

    def _pipe(k2, carry):
        base = 2 * k2
        _stage_p(base, 0)
        _stage_g(base + 1, 1)
        _stage_a(base + 2, 0)
        _stage_p(base + 1, 1)
        _stage_g(base + 2, 0)
        _stage_a(base + 3, 1)
        return carry
    lax.fori_loop(0, NP1, _pipe, 0)
    plsc.subcore_barrier()

    roff = (c * NT + R) * N
    pltpu.sync_copy(bp_hbm.at[c], bpv)

    for b in range(DPT):
        g = b * NS + s

        @pl.when(g < NDC)
        def _():
            st = pl.multiple_of(g * DB, 8)
            pltpu.sync_copy(agg_sh.at[pl.ds(st, DB), :],
                            rows0.at[pl.ds(0, DB), :])
            pltpu.sync_copy(h_hbm.at[pl.ds(roff + st, DB), :],
                            rows1.at[pl.ds(0, DB), :])

            def _radd(rr, carry2):
                for v in range(DQ // L):
                    sl = pl.ds(v * L, L)
                    rows0[rr, sl] = (rows0[rr, sl] + rows1[rr, sl]
                                     + bpv[0, sl])
                return carry2
            lax.fori_loop(0, DB, _radd, 0)
            pltpu.sync_copy(rows0.at[pl.ds(0, DB), :],
                            out_hbm.at[pl.ds(c * N + st, DB), :])


def _conv2_sc(src_hbm, dst_hbm, tab_hbm, z2d_hbm, out_hbm,
              srcv0, srcv1, dstv0, dstv1, rows0, rows1,
              agg_sh, sem_ld0, sem_ld1, sem_g0, sem_g1):
    c = lax.axis_index("c")
    s = lax.axis_index("s")
    wid = s * NC + c

    bufs = (
        (srcv0, dstv0, rows0, sem_ld0, sem_g0),
        (srcv1, dstv1, rows1, sem_ld1, sem_g1),
    )

    pltpu.sync_copy(z2d_hbm, rows0.at[pl.ds(0, DB), :])
    for b in range(DPT):
        g = b * NS + s

        @pl.when(g < NDC)
        def _():
            st = pl.multiple_of(g * DB, 8)
            pltpu.sync_copy(rows0.at[pl.ds(0, DB), :],
                            agg_sh.at[pl.ds(st, DB), :])
    plsc.subcore_barrier()

    def _stage_a(k, b):
        g = k * NW2 + wid

        @pl.when(g < NCHG)
        def _():
            srcv, dstv, rows, s_ld, s_g = bufs[b]
            off = g * CH
            pltpu.async_copy(src_hbm.at[pl.ds(off, CH)], srcv, s_ld)
            pltpu.async_copy(dst_hbm.at[pl.ds(off, CH)], dstv, s_ld)

    def _stage_g(k, b):
        g = k * NW2 + wid

        @pl.when(g < NCHG)
        def _():
            srcv, dstv, rows, s_ld, s_g = bufs[b]
            off = g * CH
            pltpu.make_async_copy(src_hbm.at[pl.ds(off, CH)], srcv,
                                  s_ld).wait()
            pltpu.make_async_copy(dst_hbm.at[pl.ds(off, CH)], dstv,
                                  s_ld).wait()
            pltpu.async_copy(tab_hbm.at[srcv], rows, s_g)

    def _stage_p(k, b):
        g = k * NW2 + wid

        @pl.when(g < NCHG)
        def _():
            srcv, dstv, rows, s_ld, s_g = bufs[b]
            pltpu.make_async_copy(tab_hbm.at[srcv], rows, s_g).wait()
            pltpu.sync_copy(rows, agg_sh.at[dstv], add=True)

    _stage_a(0, 0)
    _stage_a(1, 1)
    _stage_g(0, 0)

    def _pipe(k2, carry):
        base = 2 * k2
        _stage_p(base, 0)
        _stage_g(base + 1, 1)
        _stage_a(base + 2, 0)
        _stage_p(base + 1, 1)
        _stage_g(base + 2, 0)
        _stage_a(base + 3, 1)
        return carry
    lax.fori_loop(0, NP2, _pipe, 0)
    plsc.subcore_barrier()

    for b in range(DPT):
        g = b * NS + s

        @pl.when(g < NDC)
        def _():
            st = pl.multiple_of(g * DB, 8)
            pltpu.sync_copy(agg_sh.at[pl.ds(st, DB), :],
                            rows0.at[pl.ds(0, DB), :])
            pltpu.sync_copy(rows0.at[pl.ds(0, DB), :],
                            out_hbm.at[pl.ds(c * N + st, DB), :])


_conv1_call = functools.partial(
    pl.kernel,
    out_type=jax.ShapeDtypeStruct((NC * N, DQ), jnp.float32),
    mesh=_MESH,
    compiler_params=_SC_PARAMS,
    scratch_types=[
        pltpu.VMEM((CH,), jnp.int32),
        pltpu.VMEM((CH,), jnp.int32),
        pltpu.VMEM((CH,), jnp.int32),
        pltpu.VMEM((CH,), jnp.int32),
        pltpu.VMEM((CH,), jnp.int32),
        pltpu.VMEM((CH,), jnp.int32),
        pltpu.VMEM((CH,), jnp.int32),
        pltpu.VMEM((CH,), jnp.int32),
        pltpu.VMEM((CH,), jnp.int32),
        pltpu.VMEM((CH,), jnp.int32),
        pltpu.VMEM((CH,), jnp.float32),
        pltpu.VMEM((CH,), jnp.float32),
        pltpu.VMEM((CH,), jnp.float32),
        pltpu.VMEM((CH, DQ), jnp.float32),
        pltpu.VMEM((CH, DQ), jnp.float32),
        pltpu.VMEM((CZB,), jnp.float32),
        pltpu.VMEM((8, DQ), jnp.float32),
        pltpu.VMEM_SHARED((N, DQ), jnp.float32),
        pltpu.VMEM_SHARED((CNT,), jnp.float32),
        pltpu.SemaphoreType.DMA,
        pltpu.SemaphoreType.DMA,
        pltpu.SemaphoreType.DMA,
        pltpu.SemaphoreType.DMA,
        pltpu.SemaphoreType.DMA,
        pltpu.SemaphoreType.DMA,
    ],
)(_conv1_sc)

_conv2_call = functools.partial(
    pl.kernel,
    out_type=jax.ShapeDtypeStruct((NC * N, DQ), jnp.float32),
    mesh=_MESH,
    compiler_params=_SC_PARAMS,
    scratch_types=[
        pltpu.VMEM((CH,), jnp.int32),
        pltpu.VMEM((CH,), jnp.int32),
        pltpu.VMEM((CH,), jnp.int32),
        pltpu.VMEM((CH,), jnp.int32),
        pltpu.VMEM((CH, DQ), jnp.float32),
        pltpu.VMEM((CH, DQ), jnp.float32),
        pltpu.VMEM_SHARED((N, DQ), jnp.float32),
        pltpu.SemaphoreType.DMA,
        pltpu.SemaphoreType.DMA,
        pltpu.SemaphoreType.DMA,
        pltpu.SemaphoreType.DMA,
    ],
)(_conv2_sc)


def kernel(node_features, edge_index, edge_norm, edge_type, basis, comp,
           root_w, bias1, rel_w, root_w2, bias2):
    del edge_norm
    f32 = jnp.float32
    src = edge_index[0]
    dst = edge_index[1]
    x = node_features

    w_all = pl.pallas_call(
        _wcomp_body,
        grid=(1,),
        in_specs=[
            pl.BlockSpec((R, NBASES), lambda j: (0, 0)),
            pl.BlockSpec((NBASES, G * G), lambda j: (0, 0)),
        ],
        out_specs=pl.BlockSpec((R, G * G), lambda j: (0, 0)),
        out_shape=jax.ShapeDtypeStruct((R, G * G), f32),
    )(comp, basis.reshape(NBASES, G * G)).reshape(R, G, G)

    pw = jnp.concatenate([rel_w, root_w2], axis=1)
    pj = jnp.stack([
        jnp.pad(pw[:, c * H2:(c + 1) * H2], ((0, 0), (0, DQ - H2)))
        for c in range(NC)
    ])

    bias1_bc = jnp.broadcast_to(bias1, (8, G))
    wp_tab, bp_tab = pl.pallas_call(
        _wp_body,
        grid=(NC * NT,),
        in_specs=[
            pl.BlockSpec((1, G, G), lambda i: (jnp.minimum(i % NT, R - 1),
                                               0, 0)),
            pl.BlockSpec((G, G), lambda i: (0, 0)),
            pl.BlockSpec((8, G), lambda i: (0, 0)),
            pl.BlockSpec((1, G, DQ), lambda i: (i // NT, 0, 0)),
        ],
        out_specs=[
            pl.BlockSpec((1, G, DQ), lambda i: (i, 0, 0)),
            pl.BlockSpec((1, 8, DQ), lambda i: (i // NT, 0, 0)),
        ],
        out_shape=[
            jax.ShapeDtypeStruct((NC * NT, G, DQ), f32),
            jax.ShapeDtypeStruct((NC, 8, DQ), f32),
        ],
    )(w_all, root_w, bias1_bc, pj)

    hp = pl.pallas_call(
        _htab_body,
        grid=(NC * NT, N // BN),
        in_specs=[
            pl.BlockSpec((BN, G), lambda i, j: (j, 0)),
            pl.BlockSpec((1, G, DQ), lambda i, j: (i, 0, 0)),
        ],
        out_specs=pl.BlockSpec((1, BN, DQ), lambda i, j: (i, j, 0)),
        out_shape=jax.ShapeDtypeStruct((NC * NT, N, DQ), f32),
    )(x, wp_tab)
    hp_flat = hp.reshape(NC * NT * N, DQ)

    z2d = jnp.zeros((DB, DQ), f32)
    z1d = jnp.zeros((CZB,), f32)

    out1p = _conv1_call(src, dst, edge_type, hp_flat, bp_tab, z2d, z1d)

    agg2 = _conv2_call(src, dst, out1p, z2d)
    agg2_r = agg2.reshape(NC, N, DQ)

    bias2_bc = jnp.broadcast_to(bias2, (8, H2))
    out1p_r = out1p.reshape(NC, N, DQ)
    out = pl.pallas_call(
        _final_body,
        grid=(N // BN,),
        in_specs=[
            pl.BlockSpec((1, BN, DQ), lambda j: (0, j, 0)),
            pl.BlockSpec((1, BN, DQ), lambda j: (1, j, 0)),
            pl.BlockSpec((1, BN, DQ), lambda j: (1, j, 0)),
            pl.BlockSpec((8, H2), lambda j: (0, 0)),
        ],
        out_specs=pl.BlockSpec((BN, H2), lambda j: (j, 0)),
        out_shape=jax.ShapeDtypeStruct((N, H2), f32),
    )(agg2_r, agg2_r, out1p_r, bias2_bc)
    return out


# speedup vs baseline: 1.0021x; 1.0021x over previous
import functools

import jax
import jax.numpy as jnp
from jax import lax
from jax.experimental import pallas as pl
from jax.experimental.pallas import tpu as pltpu
from jax.experimental.pallas import tpu_sc as plsc

N = 10000
E = 160000
G = 300
H2 = 100
R = 8
NBASES = 30
NT = R + 1

L = 16
NS = 16
NC = 2
DQ = 112
CH = 128
NCHG = E // CH
TPC1 = -(-NCHG // NS)
NP1 = 40
NW2 = NC * NS
NP2 = 20
DB = 80
NDC = N // DB
DPT = -(-NDC // NS)
CNT = R * N
CZB = 1000
BN = 1000


def _wcomp_body(comp_ref, basis_ref, out_ref):
    out_ref[...] = jnp.dot(comp_ref[...], basis_ref[...],
                           preferred_element_type=jnp.float32)


def _wp_body(w_ref, root_ref, b_ref, pj_ref, out_ref, bp_ref):
    t = pl.program_id(0) % NT

    @pl.when(t < R)
    def _():
        out_ref[0] = jnp.dot(w_ref[0], pj_ref[0],
                             preferred_element_type=jnp.float32
                             ).astype(jnp.bfloat16)

    @pl.when(t == R)
    def _():
        out_ref[0] = jnp.dot(root_ref[...], pj_ref[0],
                             preferred_element_type=jnp.float32
                             ).astype(jnp.bfloat16)

    bp_ref[0] = jnp.dot(b_ref[...], pj_ref[0],
                        preferred_element_type=jnp.float32)


def _htab_body(x_ref, w_ref, out_ref):
    out_ref[0] = jnp.dot(x_ref[...].astype(jnp.bfloat16), w_ref[0],
                         preferred_element_type=jnp.float32)


def _final_body(a0_ref, a1_ref, o1_ref, b_ref, out_ref):
    acc = a0_ref[0] + a1_ref[0] + o1_ref[0]
    out_ref[...] = acc[:, :H2] + b_ref[0]


_MESH = plsc.VectorSubcoreMesh(core_axis_name="c", subcore_axis_name="s",
                               num_cores=NC, num_subcores=NS)
_SC_PARAMS = pltpu.CompilerParams(use_tc_tiling_on_sc=False)


def _conv1_sc(src_hbm, dst_hbm, typ_hbm, h_hbm, bp_hbm, z2d_hbm, z1d_hbm,
              out_hbm,
              srcv0, srcv1, dstv0, dstv1, typv0, typv1, idxv0, idxv1,
              keyv0, keyv1, cntv0, cntv1, onesv, rows0, rows1, z1, bpv,
              agg_sh, cnt_sh, sem_ld0, sem_ld1, sem_cg0, sem_cg1,
              sem_g0, sem_g1):
    c = lax.axis_index("c")
    s = lax.axis_index("s")
    coff = c * (NT * N)

    bufs = (
        (srcv0, dstv0, typv0, idxv0, keyv0, cntv0, rows0,
         sem_ld0, sem_cg0, sem_g0),
        (srcv1, dstv1, typv1, idxv1, keyv1, cntv1, rows1,
         sem_ld1, sem_cg1, sem_g1),
    )

    pltpu.sync_copy(z2d_hbm, rows0.at[pl.ds(0, DB), :])
    pltpu.sync_copy(z1d_hbm, z1)
    for b in range(DPT):
        g = b * NS + s

        @pl.when(g < NDC)
        def _():
            st = pl.multiple_of(g * DB, 8)
            pltpu.sync_copy(rows0.at[pl.ds(0, DB), :],
                            agg_sh.at[pl.ds(st, DB), :])
    for b in range(5):
        st = pl.multiple_of(s * (5 * CZB) + b * CZB, 8)
        pltpu.sync_copy(z1, cnt_sh.at[pl.ds(st, CZB)])

    def _fill_ones(i, carry):
        onesv[pl.ds(i * L, L)] = jnp.full((L,), 1.0, jnp.float32)
        return carry
    lax.fori_loop(0, CH // L, _fill_ones, 0)
    plsc.subcore_barrier()

    kbufs = ((dstv0, typv0, keyv0, sem_ld0),
             (dstv1, typv1, keyv1, sem_ld1))

    def _cstage_a(k, b):
        g = k * NS + s

        @pl.when(g < NCHG)
        def _():
            dstv, typv, keyv, s_ld = kbufs[b]
            off = g * CH
            pltpu.async_copy(dst_hbm.at[pl.ds(off, CH)], dstv, s_ld)
            pltpu.async_copy(typ_hbm.at[pl.ds(off, CH)], typv, s_ld)

    def _cstage_p(k, b):
        g = k * NS + s

        @pl.when(g < NCHG)
        def _():
            dstv, typv, keyv, s_ld = kbufs[b]
            off = g * CH
            pltpu.make_async_copy(dst_hbm.at[pl.ds(off, CH)], dstv,
                                  s_ld).wait()
            pltpu.make_async_copy(typ_hbm.at[pl.ds(off, CH)], typv,
                                  s_ld).wait()

            def _keys(i, carry2):
                sl = pl.ds(i * L, L)
                keyv[sl] = typv[sl] * N + dstv[sl]
                return carry2
            lax.fori_loop(0, CH // L, _keys, 0)
            pltpu.sync_copy(onesv, cnt_sh.at[keyv], add=True)

    _cstage_a(0, 0)
    _cstage_a(1, 1)

    def _cpipe(k2, carry):
        base = 2 * k2
        _cstage_p(base, 0)
        _cstage_a(base + 2, 0)
        _cstage_p(base + 1, 1)
        _cstage_a(base + 3, 1)
        return carry
    lax.fori_loop(0, NP1, _cpipe, 0)
    plsc.subcore_barrier()

    def _stage_a(k, b):
        g = k * NS + s

        @pl.when(g < NCHG)
        def _():
            srcv, dstv, typv, idxv, keyv, cntv, rows, s_ld, s_cg, s_g = \
                bufs[b]
            off = g * CH
            pltpu.async_copy(src_hbm.at[pl.ds(off, CH)], srcv, s_ld)
            pltpu.async_copy(dst_hbm.at[pl.ds(off, CH)], dstv, s_ld)
            pltpu.async_copy(typ_hbm.at[pl.ds(off, CH)], typv, s_ld)

    def _stage_g(k, b):
        g = k * NS + s

        @pl.when(g < NCHG)
        def _():
            srcv, dstv, typv, idxv, keyv, cntv, rows, s_ld, s_cg, s_g = \
                bufs[b]
            off = g * CH
            pltpu.make_async_copy(src_hbm.at[pl.ds(off, CH)], srcv,
                                  s_ld).wait()
            pltpu.make_async_copy(dst_hbm.at[pl.ds(off, CH)], dstv,
                                  s_ld).wait()
            pltpu.make_async_copy(typ_hbm.at[pl.ds(off, CH)], typv,
                                  s_ld).wait()

            def _keys(i, carry2):
                sl = pl.ds(i * L, L)
                t = typv[sl]
                keyv[sl] = t * N + dstv[sl]
                idxv[sl] = coff + t * N + srcv[sl]
                return carry2
            lax.fori_loop(0, CH // L, _keys, 0)
            pltpu.async_copy(cnt_sh.at[keyv], cntv, s_cg)
            pltpu.async_copy(h_hbm.at[idxv], rows, s_g)

    def _stage_p(k, b):
        g = k * NS + s

        @pl.when(g < NCHG)
        def _():
            srcv, dstv, typv, idxv, keyv, cntv, rows, s_ld, s_cg, s_g = \
                bufs[b]
            pltpu.make_async_copy(cnt_sh.at[keyv], cntv, s_cg).wait()
            pltpu.make_async_copy(h_hbm.at[idxv], rows, s_g).wait()

            def _mul(i, carry2):
                cnt16 = cntv[pl.ds(i * L, L)]
                sc = 1.0 / jnp.maximum(cnt16, 1.0)
                for j2 in range(L):
                    s16 = jnp.take_along_axis(
                        sc, jnp.full((L,), j2, jnp.int32), axis=0)
                    row = i * L + j2
                    for v in range(DQ // L):
                        sl = pl.ds(v * L, L)
                        rows[row, sl] = rows[row, sl] * s16
                return carry2
            lax.fori_loop(0, CH // L, _mul, 0)
            pltpu.sync_copy(rows, agg_sh.at[dstv], add=True)

    _stage_a(0, 0)
    _stage_a(1, 1)
    _stage_g(0, 0)

    def _pipe(k2, carry):
        base = 2 * k2
        _stage_p(base, 0)
        _stage_g(base + 1, 1)
        _stage_a(base + 2, 0)
        _stage_p(base + 1, 1)
        _stage_g(base + 2, 0)
        _stage_a(base + 3, 1)
        return carry
    lax.fori_loop(0, NP1, _pipe, 0)
    plsc.subcore_barrier()

    roff = (c * NT + R) * N
    pltpu.sync_copy(bp_hbm.at[c], bpv)

    for b in range(DPT):
        g = b * NS + s

        @pl.when(g < NDC)
        def _():
            st = pl.multiple_of(g * DB, 8)
            pltpu.sync_copy(agg_sh.at[pl.ds(st, DB), :],
                            rows0.at[pl.ds(0, DB), :])
            pltpu.sync_copy(h_hbm.at[pl.ds(roff + st, DB), :],
                            rows1.at[pl.ds(0, DB), :])

            def _radd(rr, carry2):
                for v in range(DQ // L):
                    sl = pl.ds(v * L, L)
                    rows0[rr, sl] = (rows0[rr, sl] + rows1[rr, sl]
                                     + bpv[0, sl])
                return carry2
            lax.fori_loop(0, DB, _radd, 0)
            pltpu.sync_copy(rows0.at[pl.ds(0, DB), :],
                            out_hbm.at[pl.ds(c * N + st, DB), :])


def _conv2_sc(src_hbm, dst_hbm, tab_hbm, z2d_hbm, out_hbm,
              srcv0, srcv1, dstv0, dstv1, rows0, rows1,
              agg_sh, sem_ld0, sem_ld1, sem_g0, sem_g1):
    c = lax.axis_index("c")
    s = lax.axis_index("s")
    wid = s * NC + c

    bufs = (
        (srcv0, dstv0, rows0, sem_ld0, sem_g0),
        (srcv1, dstv1, rows1, sem_ld1, sem_g1),
    )

    pltpu.sync_copy(z2d_hbm, rows0.at[pl.ds(0, DB), :])
    for b in range(DPT):
        g = b * NS + s

        @pl.when(g < NDC)
        def _():
            st = pl.multiple_of(g * DB, 8)
            pltpu.sync_copy(rows0.at[pl.ds(0, DB), :],
                            agg_sh.at[pl.ds(st, DB), :])
    plsc.subcore_barrier()

    def _stage_a(k, b):
        g = k * NW2 + wid

        @pl.when(g < NCHG)
        def _():
            srcv, dstv, rows, s_ld, s_g = bufs[b]
            off = g * CH
            pltpu.async_copy(src_hbm.at[pl.ds(off, CH)], srcv, s_ld)
            pltpu.async_copy(dst_hbm.at[pl.ds(off, CH)], dstv, s_ld)

    def _stage_g(k, b):
        g = k * NW2 + wid

        @pl.when(g < NCHG)
        def _():
            srcv, dstv, rows, s_ld, s_g = bufs[b]
            off = g * CH
            pltpu.make_async_copy(src_hbm.at[pl.ds(off, CH)], srcv,
                                  s_ld).wait()
            pltpu.make_async_copy(dst_hbm.at[pl.ds(off, CH)], dstv,
                                  s_ld).wait()
            pltpu.async_copy(tab_hbm.at[srcv], rows, s_g)

    def _stage_p(k, b):
        g = k * NW2 + wid

        @pl.when(g < NCHG)
        def _():
            srcv, dstv, rows, s_ld, s_g = bufs[b]
            pltpu.make_async_copy(tab_hbm.at[srcv], rows, s_g).wait()
            pltpu.sync_copy(rows, agg_sh.at[dstv], add=True)

    _stage_a(0, 0)
    _stage_a(1, 1)
    _stage_g(0, 0)

    def _pipe(k2, carry):
        base = 2 * k2
        _stage_p(base, 0)
        _stage_g(base + 1, 1)
        _stage_a(base + 2, 0)
        _stage_p(base + 1, 1)
        _stage_g(base + 2, 0)
        _stage_a(base + 3, 1)
        return carry
    lax.fori_loop(0, NP2, _pipe, 0)
    plsc.subcore_barrier()

    for b in range(DPT):
        g = b * NS + s

        @pl.when(g < NDC)
        def _():
            st = pl.multiple_of(g * DB, 8)
            pltpu.sync_copy(agg_sh.at[pl.ds(st, DB), :],
                            rows0.at[pl.ds(0, DB), :])
            pltpu.sync_copy(rows0.at[pl.ds(0, DB), :],
                            out_hbm.at[pl.ds(c * N + st, DB), :])


_conv1_call = functools.partial(
    pl.kernel,
    out_type=jax.ShapeDtypeStruct((NC * N, DQ), jnp.float32),
    mesh=_MESH,
    compiler_params=_SC_PARAMS,
    scratch_types=[
        pltpu.VMEM((CH,), jnp.int32),
        pltpu.VMEM((CH,), jnp.int32),
        pltpu.VMEM((CH,), jnp.int32),
        pltpu.VMEM((CH,), jnp.int32),
        pltpu.VMEM((CH,), jnp.int32),
        pltpu.VMEM((CH,), jnp.int32),
        pltpu.VMEM((CH,), jnp.int32),
        pltpu.VMEM((CH,), jnp.int32),
        pltpu.VMEM((CH,), jnp.int32),
        pltpu.VMEM((CH,), jnp.int32),
        pltpu.VMEM((CH,), jnp.float32),
        pltpu.VMEM((CH,), jnp.float32),
        pltpu.VMEM((CH,), jnp.float32),
        pltpu.VMEM((CH, DQ), jnp.float32),
        pltpu.VMEM((CH, DQ), jnp.float32),
        pltpu.VMEM((CZB,), jnp.float32),
        pltpu.VMEM((8, DQ), jnp.float32),
        pltpu.VMEM_SHARED((N, DQ), jnp.float32),
        pltpu.VMEM_SHARED((CNT,), jnp.float32),
        pltpu.SemaphoreType.DMA,
        pltpu.SemaphoreType.DMA,
        pltpu.SemaphoreType.DMA,
        pltpu.SemaphoreType.DMA,
        pltpu.SemaphoreType.DMA,
        pltpu.SemaphoreType.DMA,
    ],
)(_conv1_sc)

_conv2_call = functools.partial(
    pl.kernel,
    out_type=jax.ShapeDtypeStruct((NC * N, DQ), jnp.float32),
    mesh=_MESH,
    compiler_params=_SC_PARAMS,
    scratch_types=[
        pltpu.VMEM((CH,), jnp.int32),
        pltpu.VMEM((CH,), jnp.int32),
        pltpu.VMEM((CH,), jnp.int32),
        pltpu.VMEM((CH,), jnp.int32),
        pltpu.VMEM((CH, DQ), jnp.float32),
        pltpu.VMEM((CH, DQ), jnp.float32),
        pltpu.VMEM_SHARED((N, DQ), jnp.float32),
        pltpu.SemaphoreType.DMA,
        pltpu.SemaphoreType.DMA,
        pltpu.SemaphoreType.DMA,
        pltpu.SemaphoreType.DMA,
    ],
)(_conv2_sc)


def kernel(node_features, edge_index, edge_norm, edge_type, basis, comp,
           root_w, bias1, rel_w, root_w2, bias2):
    del edge_norm
    f32 = jnp.float32
    src = edge_index[0]
    dst = edge_index[1]
    x = node_features

    w_all = pl.pallas_call(
        _wcomp_body,
        grid=(1,),
        in_specs=[
            pl.BlockSpec((R, NBASES), lambda j: (0, 0)),
            pl.BlockSpec((NBASES, G * G), lambda j: (0, 0)),
        ],
        out_specs=pl.BlockSpec((R, G * G), lambda j: (0, 0)),
        out_shape=jax.ShapeDtypeStruct((R, G * G), f32),
    )(comp, basis.reshape(NBASES, G * G)).reshape(R, G, G)

    pw = jnp.concatenate([rel_w, root_w2], axis=1)
    pj = jnp.stack([
        jnp.pad(pw[:, c * H2:(c + 1) * H2], ((0, 0), (0, DQ - H2)))
        for c in range(NC)
    ])

    bias1_bc = jnp.broadcast_to(bias1, (8, G))
    wp_tab, bp_tab = pl.pallas_call(
        _wp_body,
        grid=(NC * NT,),
        in_specs=[
            pl.BlockSpec((1, G, G), lambda i: (jnp.minimum(i % NT, R - 1),
                                               0, 0)),
            pl.BlockSpec((G, G), lambda i: (0, 0)),
            pl.BlockSpec((8, G), lambda i: (0, 0)),
            pl.BlockSpec((1, G, DQ), lambda i: (i // NT, 0, 0)),
        ],
        out_specs=[
            pl.BlockSpec((1, G, DQ), lambda i: (i, 0, 0)),
            pl.BlockSpec((1, 8, DQ), lambda i: (i // NT, 0, 0)),
        ],
        out_shape=[
            jax.ShapeDtypeStruct((NC * NT, G, DQ), jnp.bfloat16),
            jax.ShapeDtypeStruct((NC, 8, DQ), f32),
        ],
    )(w_all, root_w, bias1_bc, pj)

    hp = pl.pallas_call(
        _htab_body,
        grid=(NC * NT, N // BN),
        in_specs=[
            pl.BlockSpec((BN, G), lambda i, j: (j, 0)),
            pl.BlockSpec((1, G, DQ), lambda i, j: (i, 0, 0)),
        ],
        out_specs=pl.BlockSpec((1, BN, DQ), lambda i, j: (i, j, 0)),
        out_shape=jax.ShapeDtypeStruct((NC * NT, N, DQ), f32),
    )(x, wp_tab)
    hp_flat = hp.reshape(NC * NT * N, DQ)

    z2d = jnp.zeros((DB, DQ), f32)
    z1d = jnp.zeros((CZB,), f32)

    out1p = _conv1_call(src, dst, edge_type, hp_flat, bp_tab, z2d, z1d)

    agg2 = _conv2_call(src, dst, out1p, z2d)
    agg2_r = agg2.reshape(NC, N, DQ)

    bias2_bc = jnp.broadcast_to(bias2, (8, H2))
    out1p_r = out1p.reshape(NC, N, DQ)
    out = pl.pallas_call(
        _final_body,
        grid=(N // BN,),
        in_specs=[
            pl.BlockSpec((1, BN, DQ), lambda j: (0, j, 0)),
            pl.BlockSpec((1, BN, DQ), lambda j: (1, j, 0)),
            pl.BlockSpec((1, BN, DQ), lambda j: (1, j, 0)),
            pl.BlockSpec((8, H2), lambda j: (0, 0)),
        ],
        out_specs=pl.BlockSpec((BN, H2), lambda j: (j, 0)),
        out_shape=jax.ShapeDtypeStruct((N, H2), f32),
    )(agg2_r, agg2_r, out1p_r, bias2_bc)
    return out


# speedup vs baseline: 1.0270x; 1.0249x over previous
import functools

import jax
import jax.numpy as jnp
from jax import lax
from jax.experimental import pallas as pl
from jax.experimental.pallas import tpu as pltpu
from jax.experimental.pallas import tpu_sc as plsc

N = 10000
E = 160000
G = 300
H2 = 100
R = 8
NBASES = 30
NT = R + 1

L = 16
NS = 16
NC = 2
DQ = 112
CH = 128
NCHG = E // CH
TPC1 = -(-NCHG // NS)
NP1 = 40
NW2 = NC * NS
NP2 = 20
DB = 80
NDC = N // DB
DPT = -(-NDC // NS)
CNT = R * N
CZB = 1000
BN = 1000


def _wcomp_body(comp_ref, basis_ref, out_ref):
    out_ref[...] = jnp.dot(comp_ref[...], basis_ref[...],
                           preferred_element_type=jnp.float32)


def _wp_body(w_ref, root_ref, b_ref, pj_ref, out_ref, bp_ref):
    t = pl.program_id(0) % NT

    @pl.when(t < R)
    def _():
        out_ref[0] = jnp.dot(w_ref[0], pj_ref[0],
                             preferred_element_type=jnp.float32
                             ).astype(jnp.bfloat16)

    @pl.when(t == R)
    def _():
        out_ref[0] = jnp.dot(root_ref[...], pj_ref[0],
                             preferred_element_type=jnp.float32
                             ).astype(jnp.bfloat16)

    bp_ref[0] = jnp.dot(b_ref[...], pj_ref[0],
                        preferred_element_type=jnp.float32)


def _htab_body(x_ref, w_ref, out_ref):
    out_ref[0] = jnp.dot(x_ref[...].astype(jnp.bfloat16), w_ref[0],
                         preferred_element_type=jnp.float32)


def _final_body(a0_ref, a1_ref, o1_ref, b_ref, out_ref):
    acc = a0_ref[0] + a1_ref[0] + o1_ref[0]
    out_ref[...] = acc[:, :H2] + b_ref[0]


_MESH = plsc.VectorSubcoreMesh(core_axis_name="c", subcore_axis_name="s",
                               num_cores=NC, num_subcores=NS)
_SC_PARAMS = pltpu.CompilerParams(use_tc_tiling_on_sc=False)


def _count_sc(dst_hbm, typ_hbm, z1d_hbm, out_hbm,
              dstv0, dstv1, typv0, typv1, keyv0, keyv1, onesv, z1, bounce,
              cnt_sh, sem_ld0, sem_ld1):
    c = lax.axis_index("c")
    s = lax.axis_index("s")
    wid = s * NC + c

    pltpu.sync_copy(z1d_hbm, z1)
    for b in range(5):
        st = pl.multiple_of(s * (5 * CZB) + b * CZB, 8)
        pltpu.sync_copy(z1, cnt_sh.at[pl.ds(st, CZB)])

    def _fill_ones(i, carry):
        onesv[pl.ds(i * L, L)] = jnp.full((L,), 1.0, jnp.float32)
        return carry
    lax.fori_loop(0, CH // L, _fill_ones, 0)
    plsc.subcore_barrier()

    kbufs = ((dstv0, typv0, keyv0, sem_ld0),
             (dstv1, typv1, keyv1, sem_ld1))

    def _cstage_a(k, b):
        g = k * NW2 + wid

        @pl.when(g < NCHG)
        def _():
            dstv, typv, keyv, s_ld = kbufs[b]
            off = g * CH
            pltpu.async_copy(dst_hbm.at[pl.ds(off, CH)], dstv, s_ld)
            pltpu.async_copy(typ_hbm.at[pl.ds(off, CH)], typv, s_ld)

    def _cstage_p(k, b):
        g = k * NW2 + wid

        @pl.when(g < NCHG)
        def _():
            dstv, typv, keyv, s_ld = kbufs[b]
            off = g * CH
            pltpu.make_async_copy(dst_hbm.at[pl.ds(off, CH)], dstv,
                                  s_ld).wait()
            pltpu.make_async_copy(typ_hbm.at[pl.ds(off, CH)], typv,
                                  s_ld).wait()

            def _keys(i, carry2):
                sl = pl.ds(i * L, L)
                keyv[sl] = typv[sl] * N + dstv[sl]
                return carry2
            lax.fori_loop(0, CH // L, _keys, 0)
            pltpu.sync_copy(onesv, cnt_sh.at[keyv], add=True)

    _cstage_a(0, 0)
    _cstage_a(1, 1)

    def _cpipe(k2, carry):
        base = 2 * k2
        _cstage_p(base, 0)
        _cstage_a(base + 2, 0)
        _cstage_p(base + 1, 1)
        _cstage_a(base + 3, 1)
        return carry
    lax.fori_loop(0, NP2 + 1, _cpipe, 0)
    plsc.subcore_barrier()

    for b in range(5):
        st = pl.multiple_of(s * (5 * CZB) + b * CZB, 8)
        pltpu.sync_copy(cnt_sh.at[pl.ds(st, CZB)], bounce)
        pltpu.sync_copy(bounce, out_hbm.at[pl.ds(c * CNT + st, CZB)])


def _conv1_sc(src_hbm, dst_hbm, typ_hbm, h_hbm, cnt2_hbm, bp_hbm, z2d_hbm,
              out_hbm,
              srcv0, srcv1, dstv0, dstv1, typv0, typv1, idxv0, idxv1,
              keyv0, keyv1, cntv0, cntv1, cntw0, cntw1, rows0, rows1, bpv,
              agg_sh, sem_ld0, sem_ld1, sem_cg0, sem_cg1,
              sem_g0, sem_g1):
    c = lax.axis_index("c")
    s = lax.axis_index("s")
    coff = c * (NT * N)

    bufs = (
        (srcv0, dstv0, typv0, idxv0, keyv0, cntv0, cntw0, rows0,
         sem_ld0, sem_cg0, sem_g0),
        (srcv1, dstv1, typv1, idxv1, keyv1, cntv1, cntw1, rows1,
         sem_ld1, sem_cg1, sem_g1),
    )

    pltpu.sync_copy(z2d_hbm, rows0.at[pl.ds(0, DB), :])
    for b in range(DPT):
        g = b * NS + s

        @pl.when(g < NDC)
        def _():
            st = pl.multiple_of(g * DB, 8)
            pltpu.sync_copy(rows0.at[pl.ds(0, DB), :],
                            agg_sh.at[pl.ds(st, DB), :])
    plsc.subcore_barrier()

    def _stage_a(k, b):
        g = k * NS + s

        @pl.when(g < NCHG)
        def _():
            srcv, dstv, typv, idxv, keyv, cntv, cntw, rows, \
                s_ld, s_cg, s_g = bufs[b]
            off = g * CH
            pltpu.async_copy(src_hbm.at[pl.ds(off, CH)], srcv, s_ld)
            pltpu.async_copy(dst_hbm.at[pl.ds(off, CH)], dstv, s_ld)
            pltpu.async_copy(typ_hbm.at[pl.ds(off, CH)], typv, s_ld)

    def _stage_g(k, b):
        g = k * NS + s

        @pl.when(g < NCHG)
        def _():
            srcv, dstv, typv, idxv, keyv, cntv, cntw, rows, \
                s_ld, s_cg, s_g = bufs[b]
            off = g * CH
            pltpu.make_async_copy(src_hbm.at[pl.ds(off, CH)], srcv,
                                  s_ld).wait()
            pltpu.make_async_copy(dst_hbm.at[pl.ds(off, CH)], dstv,
                                  s_ld).wait()
            pltpu.make_async_copy(typ_hbm.at[pl.ds(off, CH)], typv,
                                  s_ld).wait()

            def _keys(i, carry2):
                sl = pl.ds(i * L, L)
                t = typv[sl]
                keyv[sl] = t * N + dstv[sl]
                idxv[sl] = coff + t * N + srcv[sl]
                return carry2
            lax.fori_loop(0, CH // L, _keys, 0)
            pltpu.async_copy(cnt2_hbm.at[keyv], cntv, s_cg)
            pltpu.async_copy(h_hbm.at[idxv], rows, s_g)

            def _keys2(i, carry2):
                sl = pl.ds(i * L, L)
                srcv[sl] = keyv[sl] + CNT
                return carry2
            lax.fori_loop(0, CH // L, _keys2, 0)
            pltpu.async_copy(cnt2_hbm.at[srcv], cntw, s_cg)

    def _stage_p(k, b):
        g = k * NS + s

        @pl.when(g < NCHG)
        def _():
            srcv, dstv, typv, idxv, keyv, cntv, cntw, rows, \
                s_ld, s_cg, s_g = bufs[b]
            pltpu.make_async_copy(cnt2_hbm.at[keyv], cntv, s_cg).wait()
            pltpu.make_async_copy(cnt2_hbm.at[srcv], cntw, s_cg).wait()
            pltpu.make_async_copy(h_hbm.at[idxv], rows, s_g).wait()

            def _mul(i, carry2):
                sl16 = pl.ds(i * L, L)
                cnt16 = cntv[sl16] + cntw[sl16]
                sc = 1.0 / jnp.maximum(cnt16, 1.0)
                for j2 in range(L):
                    s16 = jnp.take_along_axis(
                        sc, jnp.full((L,), j2, jnp.int32), axis=0)
                    row = i * L + j2
                    for v in range(DQ // L):
                        sl = pl.ds(v * L, L)
                        rows[row, sl] = rows[row, sl] * s16
                return carry2
            lax.fori_loop(0, CH // L, _mul, 0)
            pltpu.sync_copy(rows, agg_sh.at[dstv], add=True)

    _stage_a(0, 0)
    _stage_a(1, 1)
    _stage_g(0, 0)

    def _pipe(k2, carry):
        base = 2 * k2
        _stage_p(base, 0)
        _stage_g(base + 1, 1)
        _stage_a(base + 2, 0)
        _stage_p(base + 1, 1)
        _stage_g(base + 2, 0)
        _stage_a(base + 3, 1)
        return carry
    lax.fori_loop(0, NP1, _pipe, 0)
    plsc.subcore_barrier()

    roff = (c * NT + R) * N
    pltpu.sync_copy(bp_hbm.at[c], bpv)

    for b in range(DPT):
        g = b * NS + s

        @pl.when(g < NDC)
        def _():
            st = pl.multiple_of(g * DB, 8)
            pltpu.sync_copy(agg_sh.at[pl.ds(st, DB), :],
                            rows0.at[pl.ds(0, DB), :])
            pltpu.sync_copy(h_hbm.at[pl.ds(roff + st, DB), :],
                            rows1.at[pl.ds(0, DB), :])

            def _radd(rr, carry2):
                for v in range(DQ // L):
                    sl = pl.ds(v * L, L)
                    rows0[rr, sl] = (rows0[rr, sl] + rows1[rr, sl]
                                     + bpv[0, sl])
                return carry2
            lax.fori_loop(0, DB, _radd, 0)
            pltpu.sync_copy(rows0.at[pl.ds(0, DB), :],
                            out_hbm.at[pl.ds(c * N + st, DB), :])


def _conv2_sc(src_hbm, dst_hbm, tab_hbm, z2d_hbm, out_hbm,
              srcv0, srcv1, dstv0, dstv1, rows0, rows1,
              agg_sh, sem_ld0, sem_ld1, sem_g0, sem_g1):
    c = lax.axis_index("c")
    s = lax.axis_index("s")
    wid = s * NC + c

    bufs = (
        (srcv0, dstv0, rows0, sem_ld0, sem_g0),
        (srcv1, dstv1, rows1, sem_ld1, sem_g1),
    )

    pltpu.sync_copy(z2d_hbm, rows0.at[pl.ds(0, DB), :])
    for b in range(DPT):
        g = b * NS + s

        @pl.when(g < NDC)
        def _():
            st = pl.multiple_of(g * DB, 8)
            pltpu.sync_copy(rows0.at[pl.ds(0, DB), :],
                            agg_sh.at[pl.ds(st, DB), :])
    plsc.subcore_barrier()

    def _stage_a(k, b):
        g = k * NW2 + wid

        @pl.when(g < NCHG)
        def _():
            srcv, dstv, rows, s_ld, s_g = bufs[b]
            off = g * CH
            pltpu.async_copy(src_hbm.at[pl.ds(off, CH)], srcv, s_ld)
            pltpu.async_copy(dst_hbm.at[pl.ds(off, CH)], dstv, s_ld)

    def _stage_g(k, b):
        g = k * NW2 + wid

        @pl.when(g < NCHG)
        def _():
            srcv, dstv, rows, s_ld, s_g = bufs[b]
            off = g * CH
            pltpu.make_async_copy(src_hbm.at[pl.ds(off, CH)], srcv,
                                  s_ld).wait()
            pltpu.make_async_copy(dst_hbm.at[pl.ds(off, CH)], dstv,
                                  s_ld).wait()
            pltpu.async_copy(tab_hbm.at[srcv], rows, s_g)

    def _stage_p(k, b):
        g = k * NW2 + wid

        @pl.when(g < NCHG)
        def _():
            srcv, dstv, rows, s_ld, s_g = bufs[b]
            pltpu.make_async_copy(tab_hbm.at[srcv], rows, s_g).wait()
            pltpu.sync_copy(rows, agg_sh.at[dstv], add=True)

    _stage_a(0, 0)
    _stage_a(1, 1)
    _stage_g(0, 0)

    def _pipe(k2, carry):
        base = 2 * k2
        _stage_p(base, 0)
        _stage_g(base + 1, 1)
        _stage_a(base + 2, 0)
        _stage_p(base + 1, 1)
        _stage_g(base + 2, 0)
        _stage_a(base + 3, 1)
        return carry
    lax.fori_loop(0, NP2, _pipe, 0)
    plsc.subcore_barrier()

    for b in range(DPT):
        g = b * NS + s

        @pl.when(g < NDC)
        def _():
            st = pl.multiple_of(g * DB, 8)
            pltpu.sync_copy(agg_sh.at[pl.ds(st, DB), :],
                            rows0.at[pl.ds(0, DB), :])
            pltpu.sync_copy(rows0.at[pl.ds(0, DB), :],
                            out_hbm.at[pl.ds(c * N + st, DB), :])


_count_call = functools.partial(
    pl.kernel,
    out_type=jax.ShapeDtypeStruct((NC * CNT,), jnp.float32),
    mesh=_MESH,
    compiler_params=_SC_PARAMS,
    scratch_types=[
        pltpu.VMEM((CH,), jnp.int32),
        pltpu.VMEM((CH,), jnp.int32),
        pltpu.VMEM((CH,), jnp.int32),
        pltpu.VMEM((CH,), jnp.int32),
        pltpu.VMEM((CH,), jnp.int32),
        pltpu.VMEM((CH,), jnp.int32),
        pltpu.VMEM((CH,), jnp.float32),
        pltpu.VMEM((CZB,), jnp.float32),
        pltpu.VMEM((CZB,), jnp.float32),
        pltpu.VMEM_SHARED((CNT,), jnp.float32),
        pltpu.SemaphoreType.DMA,
        pltpu.SemaphoreType.DMA,
    ],
)(_count_sc)

_conv1_call = functools.partial(
    pl.kernel,
    out_type=jax.ShapeDtypeStruct((NC * N, DQ), jnp.float32),
    mesh=_MESH,
    compiler_params=_SC_PARAMS,
    scratch_types=[
        pltpu.VMEM((CH,), jnp.int32),
        pltpu.VMEM((CH,), jnp.int32),
        pltpu.VMEM((CH,), jnp.int32),
        pltpu.VMEM((CH,), jnp.int32),
        pltpu.VMEM((CH,), jnp.int32),
        pltpu.VMEM((CH,), jnp.int32),
        pltpu.VMEM((CH,), jnp.int32),
        pltpu.VMEM((CH,), jnp.int32),
        pltpu.VMEM((CH,), jnp.int32),
        pltpu.VMEM((CH,), jnp.int32),
        pltpu.VMEM((CH,), jnp.float32),
        pltpu.VMEM((CH,), jnp.float32),
        pltpu.VMEM((CH,), jnp.float32),
        pltpu.VMEM((CH,), jnp.float32),
        pltpu.VMEM((CH, DQ), jnp.float32),
        pltpu.VMEM((CH, DQ), jnp.float32),
        pltpu.VMEM((8, DQ), jnp.float32),
        pltpu.VMEM_SHARED((N, DQ), jnp.float32),
        pltpu.SemaphoreType.DMA,
        pltpu.SemaphoreType.DMA,
        pltpu.SemaphoreType.DMA,
        pltpu.SemaphoreType.DMA,
        pltpu.SemaphoreType.DMA,
        pltpu.SemaphoreType.DMA,
    ],
)(_conv1_sc)

_conv2_call = functools.partial(
    pl.kernel,
    out_type=jax.ShapeDtypeStruct((NC * N, DQ), jnp.float32),
    mesh=_MESH,
    compiler_params=_SC_PARAMS,
    scratch_types=[
        pltpu.VMEM((CH,), jnp.int32),
        pltpu.VMEM((CH,), jnp.int32),
        pltpu.VMEM((CH,), jnp.int32),
        pltpu.VMEM((CH,), jnp.int32),
        pltpu.VMEM((CH, DQ), jnp.float32),
        pltpu.VMEM((CH, DQ), jnp.float32),
        pltpu.VMEM_SHARED((N, DQ), jnp.float32),
        pltpu.SemaphoreType.DMA,
        pltpu.SemaphoreType.DMA,
        pltpu.SemaphoreType.DMA,
        pltpu.SemaphoreType.DMA,
    ],
)(_conv2_sc)


def kernel(node_features, edge_index, edge_norm, edge_type, basis, comp,
           root_w, bias1, rel_w, root_w2, bias2):
    del edge_norm
    f32 = jnp.float32
    src = edge_index[0]
    dst = edge_index[1]
    x = node_features

    w_all = pl.pallas_call(
        _wcomp_body,
        grid=(1,),
        in_specs=[
            pl.BlockSpec((R, NBASES), lambda j: (0, 0)),
            pl.BlockSpec((NBASES, G * G), lambda j: (0, 0)),
        ],
        out_specs=pl.BlockSpec((R, G * G), lambda j: (0, 0)),
        out_shape=jax.ShapeDtypeStruct((R, G * G), f32),
    )(comp, basis.reshape(NBASES, G * G)).reshape(R, G, G)

    pw = jnp.concatenate([rel_w, root_w2], axis=1)
    pj = jnp.stack([
        jnp.pad(pw[:, c * H2:(c + 1) * H2], ((0, 0), (0, DQ - H2)))
        for c in range(NC)
    ])

    bias1_bc = jnp.broadcast_to(bias1, (8, G))
    wp_tab, bp_tab = pl.pallas_call(
        _wp_body,
        grid=(NC * NT,),
        in_specs=[
            pl.BlockSpec((1, G, G), lambda i: (jnp.minimum(i % NT, R - 1),
                                               0, 0)),
            pl.BlockSpec((G, G), lambda i: (0, 0)),
            pl.BlockSpec((8, G), lambda i: (0, 0)),
            pl.BlockSpec((1, G, DQ), lambda i: (i // NT, 0, 0)),
        ],
        out_specs=[
            pl.BlockSpec((1, G, DQ), lambda i: (i, 0, 0)),
            pl.BlockSpec((1, 8, DQ), lambda i: (i // NT, 0, 0)),
        ],
        out_shape=[
            jax.ShapeDtypeStruct((NC * NT, G, DQ), jnp.bfloat16),
            jax.ShapeDtypeStruct((NC, 8, DQ), f32),
        ],
    )(w_all, root_w, bias1_bc, pj)

    hp = pl.pallas_call(
        _htab_body,
        grid=(NC * NT, N // BN),
        in_specs=[
            pl.BlockSpec((BN, G), lambda i, j: (j, 0)),
            pl.BlockSpec((1, G, DQ), lambda i, j: (i, 0, 0)),
        ],
        out_specs=pl.BlockSpec((1, BN, DQ), lambda i, j: (i, j, 0)),
        out_shape=jax.ShapeDtypeStruct((NC * NT, N, DQ), f32),
    )(x, wp_tab)
    hp_flat = hp.reshape(NC * NT * N, DQ)

    z2d = jnp.zeros((DB, DQ), f32)
    z1d = jnp.zeros((CZB,), f32)

    cnt2 = _count_call(dst, edge_type, z1d)

    out1p = _conv1_call(src, dst, edge_type, hp_flat, cnt2, bp_tab, z2d)

    agg2 = _conv2_call(src, dst, out1p, z2d)
    agg2_r = agg2.reshape(NC, N, DQ)

    bias2_bc = jnp.broadcast_to(bias2, (8, H2))
    out1p_r = out1p.reshape(NC, N, DQ)
    out = pl.pallas_call(
        _final_body,
        grid=(N // BN,),
        in_specs=[
            pl.BlockSpec((1, BN, DQ), lambda j: (0, j, 0)),
            pl.BlockSpec((1, BN, DQ), lambda j: (1, j, 0)),
            pl.BlockSpec((1, BN, DQ), lambda j: (1, j, 0)),
            pl.BlockSpec((8, H2), lambda j: (0, 0)),
        ],
        out_specs=pl.BlockSpec((BN, H2), lambda j: (j, 0)),
        out_shape=jax.ShapeDtypeStruct((N, H2), f32),
    )(agg2_r, agg2_r, out1p_r, bias2_bc)
    return out


# speedup vs baseline: 1.0317x; 1.0045x over previous
import functools

import jax
import jax.numpy as jnp
from jax import lax
from jax.experimental import pallas as pl
from jax.experimental.pallas import tpu as pltpu
from jax.experimental.pallas import tpu_sc as plsc

N = 10000
E = 160000
G = 300
H2 = 100
R = 8
NBASES = 30
NT = R + 1

L = 16
NS = 16
NC = 2
DQ = 112
CH = 128
NCHG = E // CH
TPC1 = -(-NCHG // NS)
NP1 = 40
NW2 = NC * NS
NP2 = 20
DB = 80
NDC = N // DB
DPT = -(-NDC // NS)
CNT = R * N
CZB = 1000
BN = 1000


def _wcomp_body(comp_ref, basis_ref, out_ref):
    out_ref[...] = jnp.dot(comp_ref[...], basis_ref[...],
                           preferred_element_type=jnp.float32)


def _tab_body(w_ref, root_ref, b_ref, pj_ref, x_ref, hp_ref, bp_ref,
              wp_scr):
    t = pl.program_id(0) % NT
    j = pl.program_id(1)

    @pl.when(j == 0)
    def _():
        @pl.when(t < R)
        def _():
            wp_scr[...] = jnp.dot(w_ref[0], pj_ref[0],
                                  preferred_element_type=jnp.float32
                                  ).astype(jnp.bfloat16)

        @pl.when(t == R)
        def _():
            wp_scr[...] = jnp.dot(root_ref[...], pj_ref[0],
                                  preferred_element_type=jnp.float32
                                  ).astype(jnp.bfloat16)

        bp_ref[0] = jnp.dot(b_ref[...], pj_ref[0],
                            preferred_element_type=jnp.float32)

    @pl.when(j > 0)
    def _():
        hp_ref[0] = jnp.dot(x_ref[...].astype(jnp.bfloat16), wp_scr[...],
                            preferred_element_type=jnp.float32)


def _final_body(a0_ref, a1_ref, o1_ref, b_ref, out_ref):
    acc = a0_ref[0] + a1_ref[0] + o1_ref[0]
    out_ref[...] = acc[:, :H2] + b_ref[0]


_MESH = plsc.VectorSubcoreMesh(core_axis_name="c", subcore_axis_name="s",
                               num_cores=NC, num_subcores=NS)
_SC_PARAMS = pltpu.CompilerParams(use_tc_tiling_on_sc=False)


def _count_sc(dst_hbm, typ_hbm, z1d_hbm, out_hbm,
              dstv0, dstv1, typv0, typv1, keyv0, keyv1, onesv, z1, bounce,
              cnt_sh, sem_ld0, sem_ld1):
    c = lax.axis_index("c")
    s = lax.axis_index("s")
    wid = s * NC + c

    pltpu.sync_copy(z1d_hbm, z1)
    for b in range(5):
        st = pl.multiple_of(s * (5 * CZB) + b * CZB, 8)
        pltpu.sync_copy(z1, cnt_sh.at[pl.ds(st, CZB)])

    def _fill_ones(i, carry):
        onesv[pl.ds(i * L, L)] = jnp.full((L,), 1.0, jnp.float32)
        return carry
    lax.fori_loop(0, CH // L, _fill_ones, 0)
    plsc.subcore_barrier()

    kbufs = ((dstv0, typv0, keyv0, sem_ld0),
             (dstv1, typv1, keyv1, sem_ld1))

    def _cstage_a(k, b):
        g = k * NW2 + wid

        @pl.when(g < NCHG)
        def _():
            dstv, typv, keyv, s_ld = kbufs[b]
            off = g * CH
            pltpu.async_copy(dst_hbm.at[pl.ds(off, CH)], dstv, s_ld)
            pltpu.async_copy(typ_hbm.at[pl.ds(off, CH)], typv, s_ld)

    def _cstage_p(k, b):
        g = k * NW2 + wid

        @pl.when(g < NCHG)
        def _():
            dstv, typv, keyv, s_ld = kbufs[b]
            off = g * CH
            pltpu.make_async_copy(dst_hbm.at[pl.ds(off, CH)], dstv,
                                  s_ld).wait()
            pltpu.make_async_copy(typ_hbm.at[pl.ds(off, CH)], typv,
                                  s_ld).wait()

            def _keys(i, carry2):
                sl = pl.ds(i * L, L)
                keyv[sl] = typv[sl] * N + dstv[sl]
                return carry2
            lax.fori_loop(0, CH // L, _keys, 0)
            pltpu.sync_copy(onesv, cnt_sh.at[keyv], add=True)

    _cstage_a(0, 0)
    _cstage_a(1, 1)

    def _cpipe(k2, carry):
        base = 2 * k2
        _cstage_p(base, 0)
        _cstage_a(base + 2, 0)
        _cstage_p(base + 1, 1)
        _cstage_a(base + 3, 1)
        return carry
    lax.fori_loop(0, NP2 + 1, _cpipe, 0)
    plsc.subcore_barrier()

    for b in range(5):
        st = pl.multiple_of(s * (5 * CZB) + b * CZB, 8)
        pltpu.sync_copy(cnt_sh.at[pl.ds(st, CZB)], bounce)
        pltpu.sync_copy(bounce, out_hbm.at[pl.ds(c * CNT + st, CZB)])


def _conv1_sc(src_hbm, dst_hbm, typ_hbm, h_hbm, cnt2_hbm, bp_hbm, z2d_hbm,
              out_hbm,
              srcv0, srcv1, dstv0, dstv1, typv0, typv1, idxv0, idxv1,
              keyv0, keyv1, cntv0, cntv1, cntw0, cntw1, rows0, rows1, bpv,
              agg_sh, sem_ld0, sem_ld1, sem_cg0, sem_cg1,
              sem_g0, sem_g1):
    c = lax.axis_index("c")
    s = lax.axis_index("s")
    coff = c * (NT * N)

    bufs = (
        (srcv0, dstv0, typv0, idxv0, keyv0, cntv0, cntw0, rows0,
         sem_ld0, sem_cg0, sem_g0),
        (srcv1, dstv1, typv1, idxv1, keyv1, cntv1, cntw1, rows1,
         sem_ld1, sem_cg1, sem_g1),
    )

    pltpu.sync_copy(z2d_hbm, rows0.at[pl.ds(0, DB), :])
    for b in range(DPT):
        g = b * NS + s

        @pl.when(g < NDC)
        def _():
            st = pl.multiple_of(g * DB, 8)
            pltpu.sync_copy(rows0.at[pl.ds(0, DB), :],
                            agg_sh.at[pl.ds(st, DB), :])
    plsc.subcore_barrier()

    def _stage_a(k, b):
        g = k * NS + s

        @pl.when(g < NCHG)
        def _():
            srcv, dstv, typv, idxv, keyv, cntv, cntw, rows, \
                s_ld, s_cg, s_g = bufs[b]
            off = g * CH
            pltpu.async_copy(src_hbm.at[pl.ds(off, CH)], srcv, s_ld)
            pltpu.async_copy(dst_hbm.at[pl.ds(off, CH)], dstv, s_ld)
            pltpu.async_copy(typ_hbm.at[pl.ds(off, CH)], typv, s_ld)

    def _stage_g(k, b):
        g = k * NS + s

        @pl.when(g < NCHG)
        def _():
            srcv, dstv, typv, idxv, keyv, cntv, cntw, rows, \
                s_ld, s_cg, s_g = bufs[b]
            off = g * CH
            pltpu.make_async_copy(src_hbm.at[pl.ds(off, CH)], srcv,
                                  s_ld).wait()
            pltpu.make_async_copy(dst_hbm.at[pl.ds(off, CH)], dstv,
                                  s_ld).wait()
            pltpu.make_async_copy(typ_hbm.at[pl.ds(off, CH)], typv,
                                  s_ld).wait()

            def _keys(i, carry2):
                sl = pl.ds(i * L, L)
                t = typv[sl]
                keyv[sl] = t * N + dstv[sl]
                idxv[sl] = coff + t * N + srcv[sl]
                return carry2
            lax.fori_loop(0, CH // L, _keys, 0)
            pltpu.async_copy(cnt2_hbm.at[keyv], cntv, s_cg)
            pltpu.async_copy(h_hbm.at[idxv], rows, s_g)

            def _keys2(i, carry2):
                sl = pl.ds(i * L, L)
                srcv[sl] = keyv[sl] + CNT
                return carry2
            lax.fori_loop(0, CH // L, _keys2, 0)
            pltpu.async_copy(cnt2_hbm.at[srcv], cntw, s_cg)

    def _stage_p(k, b):
        g = k * NS + s

        @pl.when(g < NCHG)
        def _():
            srcv, dstv, typv, idxv, keyv, cntv, cntw, rows, \
                s_ld, s_cg, s_g = bufs[b]
            pltpu.make_async_copy(cnt2_hbm.at[keyv], cntv, s_cg).wait()
            pltpu.make_async_copy(cnt2_hbm.at[srcv], cntw, s_cg).wait()
            pltpu.make_async_copy(h_hbm.at[idxv], rows, s_g).wait()

            def _mul(i, carry2):
                sl16 = pl.ds(i * L, L)
                cnt16 = cntv[sl16] + cntw[sl16]
                sc = 1.0 / jnp.maximum(cnt16, 1.0)
                for j2 in range(L):
                    s16 = jnp.take_along_axis(
                        sc, jnp.full((L,), j2, jnp.int32), axis=0)
                    row = i * L + j2
                    for v in range(DQ // L):
                        sl = pl.ds(v * L, L)
                        rows[row, sl] = rows[row, sl] * s16
                return carry2
            lax.fori_loop(0, CH // L, _mul, 0)
            pltpu.sync_copy(rows, agg_sh.at[dstv], add=True)

    _stage_a(0, 0)
    _stage_a(1, 1)
    _stage_g(0, 0)

    def _pipe(k2, carry):
        base = 2 * k2
        _stage_p(base, 0)
        _stage_g(base + 1, 1)
        _stage_a(base + 2, 0)
        _stage_p(base + 1, 1)
        _stage_g(base + 2, 0)
        _stage_a(base + 3, 1)
        return carry
    lax.fori_loop(0, NP1, _pipe, 0)
    plsc.subcore_barrier()

    roff = (c * NT + R) * N
    pltpu.sync_copy(bp_hbm.at[c], bpv)

    for b in range(DPT):
        g = b * NS + s

        @pl.when(g < NDC)
        def _():
            st = pl.multiple_of(g * DB, 8)
            pltpu.sync_copy(agg_sh.at[pl.ds(st, DB), :],
                            rows0.at[pl.ds(0, DB), :])
            pltpu.sync_copy(h_hbm.at[pl.ds(roff + st, DB), :],
                            rows1.at[pl.ds(0, DB), :])

            def _radd(rr, carry2):
                for v in range(DQ // L):
                    sl = pl.ds(v * L, L)
                    rows0[rr, sl] = (rows0[rr, sl] + rows1[rr, sl]
                                     + bpv[0, sl])
                return carry2
            lax.fori_loop(0, DB, _radd, 0)
            pltpu.sync_copy(rows0.at[pl.ds(0, DB), :],
                            out_hbm.at[pl.ds(c * N + st, DB), :])


def _conv2_sc(src_hbm, dst_hbm, tab_hbm, z2d_hbm, out_hbm,
              srcv0, srcv1, dstv0, dstv1, rows0, rows1,
              agg_sh, sem_ld0, sem_ld1, sem_g0, sem_g1):
    c = lax.axis_index("c")
    s = lax.axis_index("s")
    wid = s * NC + c

    bufs = (
        (srcv0, dstv0, rows0, sem_ld0, sem_g0),
        (srcv1, dstv1, rows1, sem_ld1, sem_g1),
    )

    pltpu.sync_copy(z2d_hbm, rows0.at[pl.ds(0, DB), :])
    for b in range(DPT):
        g = b * NS + s

        @pl.when(g < NDC)
        def _():
            st = pl.multiple_of(g * DB, 8)
            pltpu.sync_copy(rows0.at[pl.ds(0, DB), :],
                            agg_sh.at[pl.ds(st, DB), :])
    plsc.subcore_barrier()

    def _stage_a(k, b):
        g = k * NW2 + wid

        @pl.when(g < NCHG)
        def _():
            srcv, dstv, rows, s_ld, s_g = bufs[b]
            off = g * CH
            pltpu.async_copy(src_hbm.at[pl.ds(off, CH)], srcv, s_ld)
            pltpu.async_copy(dst_hbm.at[pl.ds(off, CH)], dstv, s_ld)

    def _stage_g(k, b):
        g = k * NW2 + wid

        @pl.when(g < NCHG)
        def _():
            srcv, dstv, rows, s_ld, s_g = bufs[b]
            off = g * CH
            pltpu.make_async_copy(src_hbm.at[pl.ds(off, CH)], srcv,
                                  s_ld).wait()
            pltpu.make_async_copy(dst_hbm.at[pl.ds(off, CH)], dstv,
                                  s_ld).wait()
            pltpu.async_copy(tab_hbm.at[srcv], rows, s_g)

    def _stage_p(k, b):
        g = k * NW2 + wid

        @pl.when(g < NCHG)
        def _():
            srcv, dstv, rows, s_ld, s_g = bufs[b]
            pltpu.make_async_copy(tab_hbm.at[srcv], rows, s_g).wait()
            pltpu.sync_copy(rows, agg_sh.at[dstv], add=True)

    _stage_a(0, 0)
    _stage_a(1, 1)
    _stage_g(0, 0)

    def _pipe(k2, carry):
        base = 2 * k2
        _stage_p(base, 0)
        _stage_g(base + 1, 1)
        _stage_a(base + 2, 0)
        _stage_p(base + 1, 1)
        _stage_g(base + 2, 0)
        _stage_a(base + 3, 1)
        return carry
    lax.fori_loop(0, NP2, _pipe, 0)
    plsc.subcore_barrier()

    for b in range(DPT):
        g = b * NS + s

        @pl.when(g < NDC)
        def _():
            st = pl.multiple_of(g * DB, 8)
            pltpu.sync_copy(agg_sh.at[pl.ds(st, DB), :],
                            rows0.at[pl.ds(0, DB), :])
            pltpu.sync_copy(rows0.at[pl.ds(0, DB), :],
                            out_hbm.at[pl.ds(c * N + st, DB), :])


_count_call = functools.partial(
    pl.kernel,
    out_type=jax.ShapeDtypeStruct((NC * CNT,), jnp.float32),
    mesh=_MESH,
    compiler_params=_SC_PARAMS,
    scratch_types=[
        pltpu.VMEM((CH,), jnp.int32),
        pltpu.VMEM((CH,), jnp.int32),
        pltpu.VMEM((CH,), jnp.int32),
        pltpu.VMEM((CH,), jnp.int32),
        pltpu.VMEM((CH,), jnp.int32),
        pltpu.VMEM((CH,), jnp.int32),
        pltpu.VMEM((CH,), jnp.float32),
        pltpu.VMEM((CZB,), jnp.float32),
        pltpu.VMEM((CZB,), jnp.float32),
        pltpu.VMEM_SHARED((CNT,), jnp.float32),
        pltpu.SemaphoreType.DMA,
        pltpu.SemaphoreType.DMA,
    ],
)(_count_sc)

_conv1_call = functools.partial(
    pl.kernel,
    out_type=jax.ShapeDtypeStruct((NC * N, DQ), jnp.float32),
    mesh=_MESH,
    compiler_params=_SC_PARAMS,
    scratch_types=[
        pltpu.VMEM((CH,), jnp.int32),
        pltpu.VMEM((CH,), jnp.int32),
        pltpu.VMEM((CH,), jnp.int32),
        pltpu.VMEM((CH,), jnp.int32),
        pltpu.VMEM((CH,), jnp.int32),
        pltpu.VMEM((CH,), jnp.int32),
        pltpu.VMEM((CH,), jnp.int32),
        pltpu.VMEM((CH,), jnp.int32),
        pltpu.VMEM((CH,), jnp.int32),
        pltpu.VMEM((CH,), jnp.int32),
        pltpu.VMEM((CH,), jnp.float32),
        pltpu.VMEM((CH,), jnp.float32),
        pltpu.VMEM((CH,), jnp.float32),
        pltpu.VMEM((CH,), jnp.float32),
        pltpu.VMEM((CH, DQ), jnp.float32),
        pltpu.VMEM((CH, DQ), jnp.float32),
        pltpu.VMEM((8, DQ), jnp.float32),
        pltpu.VMEM_SHARED((N, DQ), jnp.float32),
        pltpu.SemaphoreType.DMA,
        pltpu.SemaphoreType.DMA,
        pltpu.SemaphoreType.DMA,
        pltpu.SemaphoreType.DMA,
        pltpu.SemaphoreType.DMA,
        pltpu.SemaphoreType.DMA,
    ],
)(_conv1_sc)

_conv2_call = functools.partial(
    pl.kernel,
    out_type=jax.ShapeDtypeStruct((NC * N, DQ), jnp.float32),
    mesh=_MESH,
    compiler_params=_SC_PARAMS,
    scratch_types=[
        pltpu.VMEM((CH,), jnp.int32),
        pltpu.VMEM((CH,), jnp.int32),
        pltpu.VMEM((CH,), jnp.int32),
        pltpu.VMEM((CH,), jnp.int32),
        pltpu.VMEM((CH, DQ), jnp.float32),
        pltpu.VMEM((CH, DQ), jnp.float32),
        pltpu.VMEM_SHARED((N, DQ), jnp.float32),
        pltpu.SemaphoreType.DMA,
        pltpu.SemaphoreType.DMA,
        pltpu.SemaphoreType.DMA,
        pltpu.SemaphoreType.DMA,
    ],
)(_conv2_sc)


def kernel(node_features, edge_index, edge_norm, edge_type, basis, comp,
           root_w, bias1, rel_w, root_w2, bias2):
    del edge_norm
    f32 = jnp.float32
    src = edge_index[0]
    dst = edge_index[1]
    x = node_features

    w_all = pl.pallas_call(
        _wcomp_body,
        grid=(1,),
        in_specs=[
            pl.BlockSpec((R, NBASES), lambda j: (0, 0)),
            pl.BlockSpec((NBASES, G * G), lambda j: (0, 0)),
        ],
        out_specs=pl.BlockSpec((R, G * G), lambda j: (0, 0)),
        out_shape=jax.ShapeDtypeStruct((R, G * G), f32),
    )(comp, basis.reshape(NBASES, G * G)).reshape(R, G, G)

    pw = jnp.concatenate([rel_w, root_w2], axis=1)
    pj = jnp.stack([
        jnp.pad(pw[:, c * H2:(c + 1) * H2], ((0, 0), (0, DQ - H2)))
        for c in range(NC)
    ])

    bias1_bc = jnp.broadcast_to(bias1, (8, G))
    hp, bp_tab = pl.pallas_call(
        _tab_body,
        grid=(NC * NT, N // BN + 1),
        in_specs=[
            pl.BlockSpec((1, G, G),
                         lambda i, j: (jnp.minimum(i % NT, R - 1), 0, 0)),
            pl.BlockSpec((G, G), lambda i, j: (0, 0)),
            pl.BlockSpec((8, G), lambda i, j: (0, 0)),
            pl.BlockSpec((1, G, DQ), lambda i, j: (i // NT, 0, 0)),
            pl.BlockSpec((BN, G), lambda i, j: (jnp.maximum(j - 1, 0), 0)),
        ],
        out_specs=[
            pl.BlockSpec((1, BN, DQ),
                         lambda i, j: (i, jnp.maximum(j - 1, 0), 0)),
            pl.BlockSpec((1, 8, DQ), lambda i, j: (i // NT, 0, 0)),
        ],
        out_shape=[
            jax.ShapeDtypeStruct((NC * NT, N, DQ), f32),
            jax.ShapeDtypeStruct((NC, 8, DQ), f32),
        ],
        scratch_shapes=[pltpu.VMEM((G, DQ), jnp.bfloat16)],
    )(w_all, root_w, bias1_bc, pj, x)
    hp_flat = hp.reshape(NC * NT * N, DQ)

    z2d = jnp.zeros((DB, DQ), f32)
    z1d = jnp.zeros((CZB,), f32)

    cnt2 = _count_call(dst, edge_type, z1d)

    out1p = _conv1_call(src, dst, edge_type, hp_flat, cnt2, bp_tab, z2d)

    agg2 = _conv2_call(src, dst, out1p, z2d)
    agg2_r = agg2.reshape(NC, N, DQ)

    bias2_bc = jnp.broadcast_to(bias2, (8, H2))
    out1p_r = out1p.reshape(NC, N, DQ)
    out = pl.pallas_call(
        _final_body,
        grid=(N // BN,),
        in_specs=[
            pl.BlockSpec((1, BN, DQ), lambda j: (0, j, 0)),
            pl.BlockSpec((1, BN, DQ), lambda j: (1, j, 0)),
            pl.BlockSpec((1, BN, DQ), lambda j: (1, j, 0)),
            pl.BlockSpec((8, H2), lambda j: (0, 0)),
        ],
        out_specs=pl.BlockSpec((BN, H2), lambda j: (j, 0)),
        out_shape=jax.ShapeDtypeStruct((N, H2), f32),
    )(agg2_r, agg2_r, out1p_r, bias2_bc)
    return out
